# baseline (speedup 1.0000x reference)
import jax
import jax.numpy as jnp
from jax import lax
from jax.experimental import pallas as pl
from jax.experimental.pallas import tpu as pltpu
from jax.experimental.pallas import tpu_sc as plsc

B = 128
C = 32767
DEPTH = 15
NC = 2
NS = 16
LANES = 16
NW = 32
SPW = B // NW
PREF = 2048
NDEEP = DEPTH - 1 - 10


def _decode_body(o_hbm, t_hbm, w_hbm, x_hbm,
                 obuf, tbuf, deep_o, deep_t, wbuf, xbuf, sem):
    wid = lax.axis_index("s") * NC + lax.axis_index("c")
    lane = lax.iota(jnp.int32, LANES)
    row0 = pl.multiple_of((wid // 2) * 8, 8)
    samp = jnp.where(lane < SPW, lane, 0)
    r = (wid % 2) * 4 + samp

    cp0 = pltpu.async_copy(o_hbm.at[pl.ds(row0, 8), pl.ds(0, PREF)], obuf, sem)
    cp1 = pltpu.async_copy(t_hbm.at[pl.ds(row0, 8), pl.ds(0, PREF)], tbuf, sem)
    cp0.wait()
    cp1.wait()

    zf = jnp.zeros((LANES,), jnp.float32)
    wbuf[0] = zf
    wbuf[DEPTH] = zf
    xbuf[0] = zf
    xbuf[DEPTH] = zf

    l0 = plsc.load_gather(obuf, [r, jnp.zeros((LANES,), jnp.int32)])
    state = dict(cur=jnp.zeros((LANES,), jnp.int32), active=l0 > 0.0, run=zf)

    def step(lvl, gather2):
        c1 = 2 * state["cur"] + 1
        c2 = c1 + 1
        l1, t1 = gather2(c1)
        l2, t2 = gather2(c2)
        take2 = l2 > l1
        lsel = jnp.where(take2, l2, l1)
        tsel = jnp.where(take2, t2, t1)
        state["cur"] = jnp.where(take2, c2, c1)
        state["active"] = state["active"] & (lsel > 0.0)
        m = state["active"] & (tsel == 0.0)
        state["run"] = jnp.where(m, state["run"] + 1.0, 0.0)
        wbuf[lvl] = jnp.where(lane < SPW, state["run"], 0.0)
        xbuf[lvl] = lsel

    def pref_gather(c):
        return (plsc.load_gather(obuf, [r, c]), plsc.load_gather(tbuf, [r, c]))

    for lvl in range(1, 11):
        step(lvl, pref_gather)

    def wbase(c, j):
        s = (2 ** j) * c + (2 ** j - 1)
        base = jnp.right_shift(s, 7) * 128
        if j == NDEEP:
            base = jnp.minimum(base, 32512)
        return base

    cur10 = state["cur"]
    cps = []
    for k in range(SPW):
        ck = cur10[k]
        for j in range(1, NDEEP + 1):
            base = pl.multiple_of(wbase(ck, j), 128)
            slot = k * NDEEP + (j - 1)
            src = pl.ds(base, 256)
            cps.append(pltpu.async_copy(
                o_hbm.at[pl.ds(row0, 8), src], deep_o.at[slot], sem))
            cps.append(pltpu.async_copy(
                t_hbm.at[pl.ds(row0, 8), src], deep_t.at[slot], sem))
    for cp in cps:
        cp.wait()

    for lvl in range(11, DEPTH):
        j = lvl - 10
        base_vec = wbase(cur10, j)
        slot_vec = samp * NDEEP + (j - 1)

        def deep_gather(c, base_vec=base_vec, slot_vec=slot_vec):
            rel = c - base_vec
            return (plsc.load_gather(deep_o, [slot_vec, r, rel]),
                    plsc.load_gather(deep_t, [slot_vec, r, rel]))

        step(lvl, deep_gather)

    obase = pl.multiple_of(wid * (DEPTH + 1), DEPTH + 1)
    pltpu.sync_copy(wbuf, w_hbm.at[pl.ds(obase, DEPTH + 1)])
    pltpu.sync_copy(xbuf, x_hbm.at[pl.ds(obase, DEPTH + 1)])


def _make_decode():
    return pl.kernel(
        _decode_body,
        out_type=(
            jax.ShapeDtypeStruct((NW * (DEPTH + 1), LANES), jnp.float32),
            jax.ShapeDtypeStruct((NW * (DEPTH + 1), LANES), jnp.float32),
        ),
        mesh=plsc.VectorSubcoreMesh(core_axis_name="c", subcore_axis_name="s",
                                    num_cores=NC, num_subcores=NS),
        scratch_types=[
            pltpu.VMEM((8, PREF), jnp.float32),
            pltpu.VMEM((8, PREF), jnp.float32),
            pltpu.VMEM((SPW * NDEEP, 8, 256), jnp.float32),
            pltpu.VMEM((SPW * NDEEP, 8, 256), jnp.float32),
            pltpu.VMEM((DEPTH + 1, LANES), jnp.float32),
            pltpu.VMEM((DEPTH + 1, LANES), jnp.float32),
            pltpu.SemaphoreType.DMA,
        ],
        compiler_params=pltpu.CompilerParams(needs_layout_passes=False),
    )


BLK = 2048
NBLK = 16


def _sum_body(o_ref, t_ref, out_ref, acc_ref):
    i = pl.program_id(0)
    col = i * BLK + lax.broadcasted_iota(jnp.int32, (8, BLK), 1)
    mask = col < C

    @pl.when(i == 0)
    def _():
        acc_ref[...] = jnp.zeros((8, BLK), jnp.float32)

    acc = acc_ref[...]
    for rr in range(0, B, 8):
        xb = o_ref[pl.ds(rr, 8), :]
        tb = t_ref[pl.ds(rr, 8), :]
        u = jnp.exp2(jnp.abs(xb) * (-1.4426950408889634))
        sp = jnp.maximum(xb, 0.0) + 0.6931471805599453 * jnp.log2(1.0 + u)
        acc = acc + jnp.where(mask, sp - xb * tb, 0.0)
    acc_ref[...] = acc

    @pl.when(i == NBLK - 1)
    def _():
        out_ref[0, 0] = jnp.sum(acc)


_sum = pl.pallas_call(
    _sum_body,
    grid=(NBLK,),
    in_specs=[
        pl.BlockSpec((B, BLK), lambda i: (0, i)),
        pl.BlockSpec((B, BLK), lambda i: (0, i)),
    ],
    out_specs=pl.BlockSpec(memory_space=pltpu.SMEM),
    out_shape=jax.ShapeDtypeStruct((1, 1), jnp.float32),
    scratch_shapes=[pltpu.VMEM((8, BLK), jnp.float32)],
)


def _combine_body(s_ref, w_ref, x_ref, out_ref):
    wv = w_ref[...]
    xv = x_ref[...]
    spw = jnp.maximum(xv, 0.0) + jnp.log1p(jnp.exp(-jnp.abs(xv)))
    extra = jnp.sum(jnp.where(wv > 0.0, wv * spw, 0.0))
    out_ref[0, 0] = (s_ref[0, 0] + extra) / (B * C)


_combine = pl.pallas_call(
    _combine_body,
    in_specs=[
        pl.BlockSpec(memory_space=pltpu.SMEM),
        pl.BlockSpec((NW * (DEPTH + 1), LANES), lambda: (0, 0)),
        pl.BlockSpec((NW * (DEPTH + 1), LANES), lambda: (0, 0)),
    ],
    out_specs=pl.BlockSpec(memory_space=pltpu.SMEM),
    out_shape=jax.ShapeDtypeStruct((1, 1), jnp.float32),
)


@jax.jit
def kernel(outputs, targets):
    w, x = _make_decode()(outputs, targets)
    dense = _sum(outputs, targets)
    total = _combine(dense, w, x)
    return total[0, 0]


# speedup vs baseline: 1.0948x; 1.0948x over previous
import jax
import jax.numpy as jnp
from jax import lax
from jax.experimental import pallas as pl
from jax.experimental.pallas import tpu as pltpu
from jax.experimental.pallas import tpu_sc as plsc

B = 128
C = 32767
DEPTH = 15
NC = 2
NS = 16
LANES = 16
NW = 32
SPW = B // NW
PREF = 2048
NDEEP = DEPTH - 1 - 10


def _decode_body(o_hbm, t_hbm, w_hbm, x_hbm,
                 obuf, tbuf, deep_o, deep_t, wbuf, xbuf, sem):
    wid = lax.axis_index("s") * NC + lax.axis_index("c")
    lane = lax.iota(jnp.int32, LANES)
    row0 = pl.multiple_of((wid // 2) * 8, 8)
    samp = jnp.where(lane < SPW, lane, 0)
    r = (wid % 2) * 4 + samp

    cp0 = pltpu.async_copy(o_hbm.at[pl.ds(row0, 8), pl.ds(0, PREF)], obuf, sem)
    cp1 = pltpu.async_copy(t_hbm.at[pl.ds(row0, 8), pl.ds(0, PREF)], tbuf, sem)
    cp0.wait()
    cp1.wait()

    zf = jnp.zeros((LANES,), jnp.float32)
    wbuf[0] = zf
    wbuf[DEPTH] = zf
    xbuf[0] = zf
    xbuf[DEPTH] = zf

    l0 = plsc.load_gather(obuf, [r, jnp.zeros((LANES,), jnp.int32)])
    state = dict(cur=jnp.zeros((LANES,), jnp.int32), active=l0 > 0.0, run=zf)

    def step(lvl, gather2):
        c1 = 2 * state["cur"] + 1
        c2 = c1 + 1
        l1, t1 = gather2(c1)
        l2, t2 = gather2(c2)
        take2 = l2 > l1
        lsel = jnp.where(take2, l2, l1)
        tsel = jnp.where(take2, t2, t1)
        state["cur"] = jnp.where(take2, c2, c1)
        state["active"] = state["active"] & (lsel > 0.0)
        m = state["active"] & (tsel == 0.0)
        state["run"] = jnp.where(m, state["run"] + 1.0, 0.0)
        wbuf[lvl] = jnp.where(lane < SPW, state["run"], 0.0)
        xbuf[lvl] = lsel

    def pref_gather(c):
        return (plsc.load_gather(obuf, [r, c]), plsc.load_gather(tbuf, [r, c]))

    for lvl in range(1, 11):
        step(lvl, pref_gather)

    def wbase(c, j):
        s = (2 ** j) * c + (2 ** j - 1)
        base = jnp.right_shift(s, 7) * 128
        if j == NDEEP:
            base = jnp.minimum(base, 32512)
        return base

    cur10 = state["cur"]
    cps = []
    for k in range(SPW):
        ck = cur10[k]
        for j in range(1, NDEEP + 1):
            base = pl.multiple_of(wbase(ck, j), 128)
            slot = k * NDEEP + (j - 1)
            src = pl.ds(base, 256)
            cps.append(pltpu.async_copy(
                o_hbm.at[pl.ds(row0, 8), src], deep_o.at[slot], sem))
            cps.append(pltpu.async_copy(
                t_hbm.at[pl.ds(row0, 8), src], deep_t.at[slot], sem))
    for cp in cps:
        cp.wait()

    for lvl in range(11, DEPTH):
        j = lvl - 10
        base_vec = wbase(cur10, j)
        slot_vec = samp * NDEEP + (j - 1)

        def deep_gather(c, base_vec=base_vec, slot_vec=slot_vec):
            rel = c - base_vec
            return (plsc.load_gather(deep_o, [slot_vec, r, rel]),
                    plsc.load_gather(deep_t, [slot_vec, r, rel]))

        step(lvl, deep_gather)

    obase = pl.multiple_of(wid * (DEPTH + 1), DEPTH + 1)
    pltpu.sync_copy(wbuf, w_hbm.at[pl.ds(obase, DEPTH + 1)])
    pltpu.sync_copy(xbuf, x_hbm.at[pl.ds(obase, DEPTH + 1)])


def _make_decode():
    return pl.kernel(
        _decode_body,
        out_type=(
            jax.ShapeDtypeStruct((NW * (DEPTH + 1), LANES), jnp.float32),
            jax.ShapeDtypeStruct((NW * (DEPTH + 1), LANES), jnp.float32),
        ),
        mesh=plsc.VectorSubcoreMesh(core_axis_name="c", subcore_axis_name="s",
                                    num_cores=NC, num_subcores=NS),
        scratch_types=[
            pltpu.VMEM((8, PREF), jnp.float32),
            pltpu.VMEM((8, PREF), jnp.float32),
            pltpu.VMEM((SPW * NDEEP, 8, 256), jnp.float32),
            pltpu.VMEM((SPW * NDEEP, 8, 256), jnp.float32),
            pltpu.VMEM((DEPTH + 1, LANES), jnp.float32),
            pltpu.VMEM((DEPTH + 1, LANES), jnp.float32),
            pltpu.SemaphoreType.DMA,
        ],
        compiler_params=pltpu.CompilerParams(needs_layout_passes=False),
    )


BLK = 4096
NBLK = 8


def _sum_body(o_ref, t_ref, out_ref, acc_ref):
    i = pl.program_id(0)
    col = i * BLK + lax.broadcasted_iota(jnp.int32, (8, BLK), 1)
    mask = col < C

    @pl.when(i == 0)
    def _():
        acc_ref[...] = jnp.zeros((8, BLK), jnp.float32)

    acc = acc_ref[...]
    for rr in range(0, B, 8):
        xb = o_ref[pl.ds(rr, 8), :]
        tb = t_ref[pl.ds(rr, 8), :]
        u = jnp.exp2(jnp.abs(xb) * (-1.4426950408889634))
        sp = jnp.maximum(xb, 0.0) + 0.6931471805599453 * jnp.log2(1.0 + u)
        acc = acc + jnp.where(mask, sp - xb * tb, 0.0)
    acc_ref[...] = acc

    @pl.when(i == NBLK - 1)
    def _():
        out_ref[0, 0] = jnp.sum(acc)


_sum = pl.pallas_call(
    _sum_body,
    grid=(NBLK,),
    in_specs=[
        pl.BlockSpec((B, BLK), lambda i: (0, i)),
        pl.BlockSpec((B, BLK), lambda i: (0, i)),
    ],
    out_specs=pl.BlockSpec(memory_space=pltpu.SMEM),
    out_shape=jax.ShapeDtypeStruct((1, 1), jnp.float32),
    scratch_shapes=[pltpu.VMEM((8, BLK), jnp.float32)],
)


def _combine_body(s_ref, w_ref, x_ref, out_ref):
    wv = w_ref[...]
    xv = x_ref[...]
    spw = jnp.maximum(xv, 0.0) + jnp.log1p(jnp.exp(-jnp.abs(xv)))
    extra = jnp.sum(jnp.where(wv > 0.0, wv * spw, 0.0))
    out_ref[0, 0] = (s_ref[0, 0] + extra) / (B * C)


_combine = pl.pallas_call(
    _combine_body,
    in_specs=[
        pl.BlockSpec(memory_space=pltpu.SMEM),
        pl.BlockSpec((NW * (DEPTH + 1), LANES), lambda: (0, 0)),
        pl.BlockSpec((NW * (DEPTH + 1), LANES), lambda: (0, 0)),
    ],
    out_specs=pl.BlockSpec(memory_space=pltpu.SMEM),
    out_shape=jax.ShapeDtypeStruct((1, 1), jnp.float32),
)


@jax.jit
def kernel(outputs, targets):
    w, x = _make_decode()(outputs, targets)
    dense = _sum(outputs, targets)
    total = _combine(dense, w, x)
    return total[0, 0]
